# baseline (speedup 1.0000x reference)
import functools

import jax
import jax.numpy as jnp
from jax import lax
from jax.experimental import pallas as pl
from jax.experimental.pallas import tpu as pltpu
from jax.experimental.pallas import tpu_sc as plsc

_B, _D, _T = 16384, 1024, 1000

_NC, _NS = 2, 16
_NW = _NC * _NS
_BPW = _B // _NW
_GCHUNK = 128

_sc_mesh = plsc.VectorSubcoreMesh(core_axis_name="c", subcore_axis_name="s")


@functools.partial(
    pl.kernel,
    mesh=_sc_mesh,
    out_type=jax.ShapeDtypeStruct((_B,), jnp.float32),
    scratch_types=[
        pltpu.VMEM((_BPW,), jnp.int32),
        pltpu.VMEM((_BPW,), jnp.float32),
        pltpu.SemaphoreType.DMA,
    ],
)
def _sc_gather(table_hbm, idx_hbm, out_hbm, idx_v, vals_v, sem):
    wid = lax.axis_index("s") * _NC + lax.axis_index("c")
    base = wid * _BPW
    pltpu.sync_copy(idx_hbm.at[pl.ds(base, _BPW)], idx_v)
    copies = []
    for j in range(_BPW // _GCHUNK):
        copies.append(pltpu.async_copy(
            table_hbm.at[idx_v.at[pl.ds(j * _GCHUNK, _GCHUNK)]],
            vals_v.at[pl.ds(j * _GCHUNK, _GCHUNK)],
            sem,
        ))
    for c in copies:
        c.wait()
    pltpu.sync_copy(vals_v, out_hbm.at[pl.ds(base, _BPW)])


_BLK = 1024


def _mix_body(ac_ref, x_ref, n_ref, o_ref):
    i = pl.program_id(0)
    a = jnp.transpose(ac_ref[i], (1, 0))
    sa = jnp.sqrt(a)
    sb = jnp.sqrt(1.0 - a)
    o_ref[...] = sa * x_ref[...] + sb * n_ref[...]


def kernel(x_start, t, noise, betas, alphas_cumprod):
    ac_t = _sc_gather(alphas_cumprod, t)
    ac2 = ac_t.reshape(_B // _BLK, 1, _BLK)
    return pl.pallas_call(
        _mix_body,
        grid=(_B // _BLK,),
        in_specs=[
            pl.BlockSpec((_B // _BLK, 1, _BLK), lambda i: (0, 0, 0)),
            pl.BlockSpec((_BLK, _D), lambda i: (i, 0)),
            pl.BlockSpec((_BLK, _D), lambda i: (i, 0)),
        ],
        out_specs=pl.BlockSpec((_BLK, _D), lambda i: (i, 0)),
        out_shape=jax.ShapeDtypeStruct((_B, _D), jnp.float32),
    )(ac2, x_start, noise)


# speedup vs baseline: 1.0200x; 1.0200x over previous
import functools

import jax
import jax.numpy as jnp
from jax import lax
from jax.experimental import pallas as pl
from jax.experimental.pallas import tpu as pltpu
from jax.experimental.pallas import tpu_sc as plsc

_B, _D, _T = 16384, 1024, 1000

_NC, _NS = 2, 16
_NW = _NC * _NS
_BPW = _B // _NW
_GCHUNK = 128

_sc_mesh = plsc.VectorSubcoreMesh(core_axis_name="c", subcore_axis_name="s")


@functools.partial(
    pl.kernel,
    mesh=_sc_mesh,
    out_type=jax.ShapeDtypeStruct((_B,), jnp.float32),
    scratch_types=[
        pltpu.VMEM((_BPW,), jnp.int32),
        pltpu.VMEM((_BPW,), jnp.float32),
        pltpu.SemaphoreType.DMA,
    ],
)
def _sc_gather(table_hbm, idx_hbm, out_hbm, idx_v, vals_v, sem):
    wid = lax.axis_index("s") * _NC + lax.axis_index("c")
    base = wid * _BPW
    pltpu.sync_copy(idx_hbm.at[pl.ds(base, _BPW)], idx_v)
    copies = []
    for j in range(_BPW // _GCHUNK):
        copies.append(pltpu.async_copy(
            table_hbm.at[idx_v.at[pl.ds(j * _GCHUNK, _GCHUNK)]],
            vals_v.at[pl.ds(j * _GCHUNK, _GCHUNK)],
            sem,
        ))
    for c in copies:
        c.wait()
    pltpu.sync_copy(vals_v, out_hbm.at[pl.ds(base, _BPW)])


_BLK = 1024


def _mix_body(ac_ref, x_ref, n_ref, o_ref):
    a = jnp.transpose(ac_ref[0], (1, 0))
    sa = jnp.sqrt(a)
    sb = jnp.sqrt(1.0 - a)
    o_ref[...] = sa * x_ref[...] + sb * n_ref[...]


def kernel(x_start, t, noise, betas, alphas_cumprod):
    ac_t = _sc_gather(alphas_cumprod, t)
    ac2 = ac_t.reshape(_B // _BLK, 1, _BLK)
    return pl.pallas_call(
        _mix_body,
        grid=(_B // _BLK,),
        in_specs=[
            pl.BlockSpec((1, 1, _BLK), lambda i: (i, 0, 0)),
            pl.BlockSpec((_BLK, _D), lambda i: (i, 0)),
            pl.BlockSpec((_BLK, _D), lambda i: (i, 0)),
        ],
        out_specs=pl.BlockSpec((_BLK, _D), lambda i: (i, 0)),
        out_shape=jax.ShapeDtypeStruct((_B, _D), jnp.float32),
    )(ac2, x_start, noise)
